# baseline (speedup 1.0000x reference)
import functools

import jax
import jax.numpy as jnp
from jax import lax
from jax.experimental import pallas as pl
from jax.experimental.pallas import tpu as pltpu
from jax.experimental.pallas import tpu_sc as plsc

N = 50000
E = 800000
G = 64
F_IN = 4
H = 64
C = 3
EPS = 1e-5

NC = 2
NS = 16

NPAD = 50048
RPT = NPAD // NS
KC = 256
E_PAD = 819200
CHUNKS = E_PAD // KC
CPW1 = CHUNKS // (NC * NS)
CPT2 = CHUNKS // NS

NB = 16
BR = NPAD // NB

_SC_MESH = dict(core_axis_name="c", subcore_axis_name="s")


def _sc_agg8(table, ei, zeros8):
    mesh = plsc.VectorSubcoreMesh(**_SC_MESH)

    @functools.partial(
        pl.kernel,
        out_type=(jax.ShapeDtypeStruct((NPAD, 8), jnp.float32),
                  jax.ShapeDtypeStruct((NPAD, 8), jnp.float32)),
        mesh=mesh,
        compiler_params=pltpu.CompilerParams(use_tc_tiling_on_sc=False),
        scratch_types=[
            pltpu.VMEM((2, KC), jnp.int32),
            pltpu.VMEM((2, KC), jnp.int32),
            pltpu.VMEM((KC, 8), jnp.float32),
            pltpu.VMEM((KC, 8), jnp.float32),
            pltpu.VMEM_SHARED((NPAD, 8), jnp.float32),
            pltpu.SemaphoreType.DMA,
            pltpu.SemaphoreType.DMA,
        ],
    )
    def run(table_h, ei_h, z_h, out0_h, out1_h,
            sd0, sd1, rows0, rows1, acc, sem0, sem1):
        cid = lax.axis_index("c")
        sid = lax.axis_index("s")
        wid = sid * NC + cid
        base = wid * CPW1
        pltpu.sync_copy(z_h, acc.at[pl.ds(sid * RPT, RPT)])
        plsc.subcore_barrier()

        def load(ch, sd_v):
            pltpu.sync_copy(ei_h.at[:, pl.ds(ch * KC, KC)], sd_v)

        load(base, sd0)
        pltpu.async_copy(table_h.at[sd0.at[0]], rows0, sem0)

        def body(j, carry):
            c1 = base + 2 * j + 1
            c2 = jnp.minimum(c1 + 1, base + CPW1 - 1)
            load(c1, sd1)
            pltpu.make_async_copy(table_h.at[sd0.at[0]], rows0, sem0).wait()
            pltpu.async_copy(table_h.at[sd1.at[0]], rows1, sem1)
            pltpu.sync_copy(rows0, acc.at[sd0.at[1]], add=True)
            load(c2, sd0)
            pltpu.make_async_copy(table_h.at[sd1.at[0]], rows1, sem1).wait()
            pltpu.async_copy(table_h.at[sd0.at[0]], rows0, sem0)
            pltpu.sync_copy(rows1, acc.at[sd1.at[1]], add=True)
            return carry

        lax.fori_loop(0, CPW1 // 2, body, 0)
        pltpu.make_async_copy(table_h.at[sd0.at[0]], rows0, sem0).wait()
        plsc.subcore_barrier()

        @pl.when(cid == 0)
        def _():
            pltpu.sync_copy(acc.at[pl.ds(sid * RPT, RPT)], out0_h.at[pl.ds(sid * RPT, RPT)])

        @pl.when(cid == 1)
        def _():
            pltpu.sync_copy(acc.at[pl.ds(sid * RPT, RPT)], out1_h.at[pl.ds(sid * RPT, RPT)])

    return run(table, ei, zeros8)


def _sc_agg32(t0, t1, ei, zeros32):
    mesh = plsc.VectorSubcoreMesh(**_SC_MESH)

    @functools.partial(
        pl.kernel,
        out_type=tuple(jax.ShapeDtypeStruct((NPAD, 32), jnp.float32)
                       for _ in range(2)),
        mesh=mesh,
        compiler_params=pltpu.CompilerParams(use_tc_tiling_on_sc=False),
        scratch_types=[
            pltpu.VMEM((2, KC), jnp.int32),
            pltpu.VMEM((2, KC), jnp.int32),
            pltpu.VMEM((KC, 32), jnp.float32),
            pltpu.VMEM((KC, 32), jnp.float32),
            pltpu.VMEM_SHARED((NPAD, 32), jnp.float32),
            pltpu.SemaphoreType.DMA,
            pltpu.SemaphoreType.DMA,
        ],
    )
    def run(t0_h, t1_h, ei_h, z_h, o0_h, o1_h,
            sd0, sd1, rows0, rows1, acc, sem0, sem1):
        cid = lax.axis_index("c")
        sid = lax.axis_index("s")
        row = pl.ds(sid * RPT, RPT)
        base = sid * CPT2
        pltpu.sync_copy(z_h, acc.at[row])
        plsc.subcore_barrier()

        def load(ch, sd_v):
            pltpu.sync_copy(ei_h.at[:, pl.ds(ch * KC, KC)], sd_v)

        def gstart(sd_v, r_v, sem):
            @pl.when(cid == 0)
            def _():
                pltpu.async_copy(t0_h.at[sd_v.at[0]], r_v, sem)

            @pl.when(cid == 1)
            def _():
                pltpu.async_copy(t1_h.at[sd_v.at[0]], r_v, sem)

        def gwait(sd_v, r_v, sem):
            @pl.when(cid == 0)
            def _():
                pltpu.make_async_copy(t0_h.at[sd_v.at[0]], r_v, sem).wait()

            @pl.when(cid == 1)
            def _():
                pltpu.make_async_copy(t1_h.at[sd_v.at[0]], r_v, sem).wait()

        load(base, sd0)
        gstart(sd0, rows0, sem0)

        def body(j, carry):
            c1 = base + 2 * j + 1
            c2 = jnp.minimum(c1 + 1, base + CPT2 - 1)
            load(c1, sd1)
            gwait(sd0, rows0, sem0)
            gstart(sd1, rows1, sem1)
            pltpu.sync_copy(rows0, acc.at[sd0.at[1]], add=True)
            load(c2, sd0)
            gwait(sd1, rows1, sem1)
            gstart(sd0, rows0, sem0)
            pltpu.sync_copy(rows1, acc.at[sd1.at[1]], add=True)
            return carry

        lax.fori_loop(0, CPT2 // 2, body, 0)
        gwait(sd0, rows0, sem0)
        plsc.subcore_barrier()

        @pl.when(cid == 0)
        def _():
            pltpu.sync_copy(acc.at[row], o0_h.at[row])

        @pl.when(cid == 1)
        def _():
            pltpu.sync_copy(acc.at[row], o1_h.at[row])

    return run(t0, t1, ei, zeros32)


def _rows(shape):
    return pl.BlockSpec(shape, lambda i: (i,) + (0,) * (len(shape) - 1))


def _full(shape):
    return pl.BlockSpec(shape, lambda i: (0,) * len(shape))


def _onehot(b_r):
    return (b_r[...] == lax.broadcasted_iota(jnp.int32, (BR, G), 1)
            ).astype(jnp.float32)


def _rows2(shape):
    return pl.BlockSpec(shape, lambda i: (i % NB,) + (0,) * (len(shape) - 1))


def _full2(shape):
    return pl.BlockSpec(shape, lambda i: (0,) * len(shape))


def _phase_ab1(a0, a1, table0, bseg, Wl, bl, Wr, gnw, gnb, gna):
    def body(a0_r, a1_r, t_r, b_r, wl_r, bl_r, wr_r, w_r, bb_r, ms_r,
             o0, o1, invd_r, cnt_r, s1_s, s2_s, h_s):
        i = pl.program_id(0)
        a8 = a0_r[...] + a1_r[...]
        invd = 1.0 / jnp.maximum(a8[:, 4:5], 1.0)
        invd_r[...] = invd
        m = _onehot(b_r)
        row = pl.ds((i % NB) * BR, BR)

        @pl.when(i < NB)
        def _():
            agg4 = a8[:, 0:4] * invd
            x4 = t_r[:, 0:4]
            h = (jnp.dot(agg4, wl_r[...], preferred_element_type=jnp.float32)
                 + bl_r[...]
                 + jnp.dot(x4, wr_r[...], preferred_element_type=jnp.float32))
            h_s[row, :] = h

            @pl.when(i == 0)
            def _():
                s1_s[...] = jnp.zeros_like(s1_s)
                s2_s[...] = jnp.zeros_like(s2_s)
                cnt_r[...] = jnp.zeros_like(cnt_r)

            dn = (((0,), (0,)), ((), ()))
            s1_s[...] += lax.dot_general(m, h, dn,
                                         preferred_element_type=jnp.float32)
            s2_s[...] += lax.dot_general(m, h * h, dn,
                                         preferred_element_type=jnp.float32)
            cnt_r[...] += lax.dot_general(m, jnp.ones((BR, 1), jnp.float32), dn,
                                          preferred_element_type=jnp.float32)

        @pl.when(i >= NB)
        def _():
            p, q = _graphnorm_pq(s1_s[...], s2_s[...], cnt_r[...],
                                 w_r[...], bb_r[...], ms_r[...])
            pb = jnp.dot(m, p, preferred_element_type=jnp.float32)
            qb = jnp.dot(m, q, preferred_element_type=jnp.float32)
            xn = jnp.maximum(pb * h_s[row, :] + qb, 0.0)
            o0[...] = xn[:, 0:32]
            o1[...] = xn[:, 32:64]

    return pl.pallas_call(
        body,
        grid=(2 * NB,),
        in_specs=[_rows2((BR, 8)), _rows2((BR, 8)), _rows2((BR, 8)),
                  _rows2((BR, 1)),
                  _full2((F_IN, H)), _full2((1, H)), _full2((F_IN, H)),
                  _full2((1, H)), _full2((1, H)), _full2((1, H))],
        out_specs=[_rows2((BR, 32)), _rows2((BR, 32)), _rows2((BR, 1)),
                   _full2((G, 1))],
        out_shape=[jax.ShapeDtypeStruct((NPAD, 32), jnp.float32),
                   jax.ShapeDtypeStruct((NPAD, 32), jnp.float32),
                   jax.ShapeDtypeStruct((NPAD, 1), jnp.float32),
                   jax.ShapeDtypeStruct((G, 1), jnp.float32)],
        scratch_shapes=[pltpu.VMEM((G, H), jnp.float32),
                        pltpu.VMEM((G, H), jnp.float32),
                        pltpu.VMEM((NPAD, H), jnp.float32)],
    )(a0, a1, table0, bseg, Wl, bl, Wr, gnw, gnb, gna)


def _phase_ab23(xq, aq, invd, bseg, cnt, Wl, bl, Wr, gnw, gnb, gna):
    def body(x0, x1, a0, a1, iv_r, b_r, cnt_r, wl_r, bl_r, wr_r,
             w_r, bb_r, ms_r, o0, o1, s1_s, s2_s, h_s):
        i = pl.program_id(0)
        m = _onehot(b_r)
        row = pl.ds((i % NB) * BR, BR)

        @pl.when(i < NB)
        def _():
            x = jnp.concatenate([x0[...], x1[...]], axis=1)
            agg = jnp.concatenate([a0[...], a1[...]], axis=1) * iv_r[...]
            h = (jnp.dot(agg, wl_r[...], preferred_element_type=jnp.float32)
                 + bl_r[...]
                 + jnp.dot(x, wr_r[...], preferred_element_type=jnp.float32))
            h_s[row, :] = h

            @pl.when(i == 0)
            def _():
                s1_s[...] = jnp.zeros_like(s1_s)
                s2_s[...] = jnp.zeros_like(s2_s)

            dn = (((0,), (0,)), ((), ()))
            s1_s[...] += lax.dot_general(m, h, dn,
                                         preferred_element_type=jnp.float32)
            s2_s[...] += lax.dot_general(m, h * h, dn,
                                         preferred_element_type=jnp.float32)

        @pl.when(i >= NB)
        def _():
            p, q = _graphnorm_pq(s1_s[...], s2_s[...], cnt_r[...],
                                 w_r[...], bb_r[...], ms_r[...])
            pb = jnp.dot(m, p, preferred_element_type=jnp.float32)
            qb = jnp.dot(m, q, preferred_element_type=jnp.float32)
            xn = pb * h_s[row, :] + qb
            xn = xn + jnp.concatenate([x0[...], x1[...]], axis=1)
            xn = jnp.maximum(xn, 0.0)
            o0[...] = xn[:, 0:32]
            o1[...] = xn[:, 32:64]

    return pl.pallas_call(
        body,
        grid=(2 * NB,),
        in_specs=[_rows2((BR, 32))] * 4 + [
                  _rows2((BR, 1)), _rows2((BR, 1)), _full2((G, 1)),
                  _full2((H, H)), _full2((1, H)), _full2((H, H)),
                  _full2((1, H)), _full2((1, H)), _full2((1, H))],
        out_specs=[_rows2((BR, 32)), _rows2((BR, 32))],
        out_shape=[jax.ShapeDtypeStruct((NPAD, 32), jnp.float32),
                   jax.ShapeDtypeStruct((NPAD, 32), jnp.float32)],
        scratch_shapes=[pltpu.VMEM((G, H), jnp.float32),
                        pltpu.VMEM((G, H), jnp.float32),
                        pltpu.VMEM((NPAD, H), jnp.float32)],
    )(*xq, *aq, invd, bseg, cnt, Wl, bl, Wr, gnw, gnb, gna)


def _graphnorm_pq(s1, s2, cnt, gnw, gnb, gna):
    rc = 1.0 / jnp.maximum(cnt, 1.0)
    mean = s1 * rc
    ms = gna
    var = s2 * rc - (2.0 - ms) * ms * mean * mean
    inv = lax.rsqrt(var + EPS)
    p = gnw * inv
    q = gnb - p * ms * mean
    return p, q


def _phase_ab3(xq, aq, invd, bseg, cnt, Wl, bl, Wr, gnw, gnb, gna, Wo, bo):
    def body(x0, x1, a0, a1, iv_r, b_r, cnt_r, wl_r, bl_r, wr_r,
             w_r, bb_r, ms_r, wo_r, bo_r, out_r, s1_s, s2_s, pool_s, h_s):
        i = pl.program_id(0)
        m = _onehot(b_r)
        row = pl.ds((i % NB) * BR, BR)
        dn = (((0,), (0,)), ((), ()))

        @pl.when(i < NB)
        def _():
            x = jnp.concatenate([x0[...], x1[...]], axis=1)
            agg = jnp.concatenate([a0[...], a1[...]], axis=1) * iv_r[...]
            h = (jnp.dot(agg, wl_r[...], preferred_element_type=jnp.float32)
                 + bl_r[...]
                 + jnp.dot(x, wr_r[...], preferred_element_type=jnp.float32))
            h_s[row, :] = h

            @pl.when(i == 0)
            def _():
                s1_s[...] = jnp.zeros_like(s1_s)
                s2_s[...] = jnp.zeros_like(s2_s)
                pool_s[...] = jnp.zeros_like(pool_s)

            s1_s[...] += lax.dot_general(m, h, dn,
                                         preferred_element_type=jnp.float32)
            s2_s[...] += lax.dot_general(m, h * h, dn,
                                         preferred_element_type=jnp.float32)

        @pl.when(i >= NB)
        def _():
            p, q = _graphnorm_pq(s1_s[...], s2_s[...], cnt_r[...],
                                 w_r[...], bb_r[...], ms_r[...])
            pb = jnp.dot(m, p, preferred_element_type=jnp.float32)
            qb = jnp.dot(m, q, preferred_element_type=jnp.float32)
            xn = (pb * h_s[row, :] + qb
                  + jnp.concatenate([x0[...], x1[...]], axis=1))
            xn = jnp.maximum(xn, 0.0)
            pool_s[...] += lax.dot_general(m, xn, dn,
                                           preferred_element_type=jnp.float32)

            @pl.when(i == 2 * NB - 1)
            def _():
                pooled = pool_s[...] / cnt_r[...]
                out_r[...] = (jnp.dot(pooled, wo_r[...],
                                      preferred_element_type=jnp.float32)
                              + bo_r[...])

    return pl.pallas_call(
        body,
        grid=(2 * NB,),
        in_specs=[_rows2((BR, 32))] * 4 + [
                  _rows2((BR, 1)), _rows2((BR, 1)), _full2((G, 1)),
                  _full2((H, H)), _full2((1, H)), _full2((H, H)),
                  _full2((1, H)), _full2((1, H)), _full2((1, H)),
                  _full2((H, C)), _full2((1, C))],
        out_specs=[_full2((G, C))],
        out_shape=[jax.ShapeDtypeStruct((G, C), jnp.float32)],
        scratch_shapes=[pltpu.VMEM((G, H), jnp.float32),
                        pltpu.VMEM((G, H), jnp.float32),
                        pltpu.VMEM((G, H), jnp.float32),
                        pltpu.VMEM((NPAD, H), jnp.float32)],
    )(*xq, *aq, invd, bseg, cnt, Wl, bl, Wr, gnw, gnb, gna, Wo, bo)[0]


def kernel(x, edge_index, batch,
           W1l, b1l, W1r, gn1_w, gn1_b, gn1_a,
           W2l, b2l, W2r, gn2_w, gn2_b, gn2_a,
           W3l, b3l, W3r, gn3_w, gn3_b, gn3_a,
           W_out, b_out):
    pad_e = E_PAD - E
    ei = jnp.stack([jnp.pad(edge_index[0], (0, pad_e)),
                    jnp.pad(edge_index[1], (0, pad_e), constant_values=N)])
    bseg = jnp.pad(batch, (0, NPAD - N), constant_values=G).reshape(NPAD, 1)
    table0 = jnp.pad(
        jnp.concatenate([x, jnp.ones((N, 1), jnp.float32)], axis=1),
        ((0, NPAD - N), (0, 8 - F_IN - 1)))
    z8 = jnp.zeros((RPT, 8), jnp.float32)
    z32 = jnp.zeros((RPT, 32), jnp.float32)

    r1 = lambda v: v.reshape(1, -1)

    a0, a1 = _sc_agg8(table0, ei, z8)
    x1a, x1b, invd, cnt = _phase_ab1(a0, a1, table0, bseg, W1l, r1(b1l), W1r,
                                     r1(gn1_w), r1(gn1_b), r1(gn1_a))
    x1q = (x1a, x1b)

    g1q = _sc_agg32(*x1q, ei, z32)
    x2q = _phase_ab23(x1q, g1q, invd, bseg, cnt, W2l, r1(b2l), W2r,
                      r1(gn2_w), r1(gn2_b), r1(gn2_a))

    g2q = _sc_agg32(*x2q, ei, z32)
    return _phase_ab3(x2q, g2q, invd, bseg, cnt, W3l, r1(b3l), W3r,
                      r1(gn3_w), r1(gn3_b), r1(gn3_a), W_out, r1(b_out))


# speedup vs baseline: 1.0813x; 1.0813x over previous
import functools

import jax
import jax.numpy as jnp
from jax import lax
from jax.experimental import pallas as pl
from jax.experimental.pallas import tpu as pltpu
from jax.experimental.pallas import tpu_sc as plsc

N = 50000
E = 800000
G = 64
F_IN = 4
H = 64
C = 3
EPS = 1e-5

NC = 2
NS = 16

NPAD = 50048
RPT = NPAD // NS
KC = 256
E_PAD = 819200
CHUNKS = E_PAD // KC
CPW1 = CHUNKS // (NC * NS)
CPT2 = CHUNKS // NS

NB = 16
BR = NPAD // NB

_SC_MESH = dict(core_axis_name="c", subcore_axis_name="s")


def _sc_agg8(table, ei, zeros8):
    mesh = plsc.VectorSubcoreMesh(**_SC_MESH)

    @functools.partial(
        pl.kernel,
        out_type=(jax.ShapeDtypeStruct((NPAD, 8), jnp.float32),
                  jax.ShapeDtypeStruct((NPAD, 8), jnp.float32)),
        mesh=mesh,
        compiler_params=pltpu.CompilerParams(use_tc_tiling_on_sc=False),
        scratch_types=[
            pltpu.VMEM((2, KC), jnp.int32),
            pltpu.VMEM((2, KC), jnp.int32),
            pltpu.VMEM((KC, 8), jnp.float32),
            pltpu.VMEM((KC, 8), jnp.float32),
            pltpu.VMEM_SHARED((NPAD, 8), jnp.float32),
            pltpu.SemaphoreType.DMA,
            pltpu.SemaphoreType.DMA,
        ],
    )
    def run(table_h, ei_h, z_h, out0_h, out1_h,
            sd0, sd1, rows0, rows1, acc, sem0, sem1):
        cid = lax.axis_index("c")
        sid = lax.axis_index("s")
        wid = sid * NC + cid
        base = wid * CPW1
        pltpu.sync_copy(z_h, acc.at[pl.ds(sid * RPT, RPT)])
        plsc.subcore_barrier()

        def load(ch, sd_v):
            pltpu.sync_copy(ei_h.at[:, pl.ds(ch * KC, KC)], sd_v)

        load(base, sd0)
        pltpu.async_copy(table_h.at[sd0.at[0]], rows0, sem0)

        def body(j, carry):
            c1 = base + 2 * j + 1
            c2 = jnp.minimum(c1 + 1, base + CPW1 - 1)
            load(c1, sd1)
            pltpu.make_async_copy(table_h.at[sd0.at[0]], rows0, sem0).wait()
            pltpu.async_copy(table_h.at[sd1.at[0]], rows1, sem1)
            pltpu.sync_copy(rows0, acc.at[sd0.at[1]], add=True)
            load(c2, sd0)
            pltpu.make_async_copy(table_h.at[sd1.at[0]], rows1, sem1).wait()
            pltpu.async_copy(table_h.at[sd0.at[0]], rows0, sem0)
            pltpu.sync_copy(rows1, acc.at[sd1.at[1]], add=True)
            return carry

        lax.fori_loop(0, CPW1 // 2, body, 0)
        pltpu.make_async_copy(table_h.at[sd0.at[0]], rows0, sem0).wait()
        plsc.subcore_barrier()

        @pl.when(cid == 0)
        def _():
            pltpu.sync_copy(acc.at[pl.ds(sid * RPT, RPT)], out0_h.at[pl.ds(sid * RPT, RPT)])

        @pl.when(cid == 1)
        def _():
            pltpu.sync_copy(acc.at[pl.ds(sid * RPT, RPT)], out1_h.at[pl.ds(sid * RPT, RPT)])

    return run(table, ei, zeros8)


def _sc_agg32(t0, t1, ei, zeros32):
    mesh = plsc.VectorSubcoreMesh(**_SC_MESH)

    @functools.partial(
        pl.kernel,
        out_type=tuple(jax.ShapeDtypeStruct((NPAD, 32), jnp.float32)
                       for _ in range(2)),
        mesh=mesh,
        compiler_params=pltpu.CompilerParams(use_tc_tiling_on_sc=False),
        scratch_types=[
            pltpu.VMEM((2, KC), jnp.int32),
            pltpu.VMEM((2, KC), jnp.int32),
            pltpu.VMEM((KC, 32), jnp.float32),
            pltpu.VMEM((KC, 32), jnp.float32),
            pltpu.VMEM_SHARED((NPAD, 32), jnp.float32),
            pltpu.SemaphoreType.DMA,
            pltpu.SemaphoreType.DMA,
        ],
    )
    def run(t0_h, t1_h, ei_h, z_h, o0_h, o1_h,
            sd0, sd1, rows0, rows1, acc, sem0, sem1):
        cid = lax.axis_index("c")
        sid = lax.axis_index("s")
        row = pl.ds(sid * RPT, RPT)
        base = sid * CPT2
        pltpu.sync_copy(z_h, acc.at[row])
        plsc.subcore_barrier()

        def load(ch, sd_v):
            pltpu.sync_copy(ei_h.at[:, pl.ds(ch * KC, KC)], sd_v)

        def gstart(sd_v, r_v, sem):
            @pl.when(cid == 0)
            def _():
                pltpu.async_copy(t0_h.at[sd_v.at[0]], r_v, sem)

            @pl.when(cid == 1)
            def _():
                pltpu.async_copy(t1_h.at[sd_v.at[0]], r_v, sem)

        def gwait(sd_v, r_v, sem):
            @pl.when(cid == 0)
            def _():
                pltpu.make_async_copy(t0_h.at[sd_v.at[0]], r_v, sem).wait()

            @pl.when(cid == 1)
            def _():
                pltpu.make_async_copy(t1_h.at[sd_v.at[0]], r_v, sem).wait()

        load(base, sd0)
        gstart(sd0, rows0, sem0)

        def body(j, carry):
            c1 = base + 2 * j + 1
            c2 = jnp.minimum(c1 + 1, base + CPT2 - 1)
            load(c1, sd1)
            gwait(sd0, rows0, sem0)
            gstart(sd1, rows1, sem1)
            pltpu.sync_copy(rows0, acc.at[sd0.at[1]], add=True)
            load(c2, sd0)
            gwait(sd1, rows1, sem1)
            gstart(sd0, rows0, sem0)
            pltpu.sync_copy(rows1, acc.at[sd1.at[1]], add=True)
            return carry

        lax.fori_loop(0, CPT2 // 2, body, 0)
        gwait(sd0, rows0, sem0)
        plsc.subcore_barrier()

        @pl.when(cid == 0)
        def _():
            pltpu.sync_copy(acc.at[row], o0_h.at[row])

        @pl.when(cid == 1)
        def _():
            pltpu.sync_copy(acc.at[row], o1_h.at[row])

    return run(t0, t1, ei, zeros32)


def _rows(shape):
    return pl.BlockSpec(shape, lambda i: (i,) + (0,) * (len(shape) - 1))


def _full(shape):
    return pl.BlockSpec(shape, lambda i: (0,) * len(shape))


def _onehot(b_r):
    return (b_r[...] == lax.broadcasted_iota(jnp.int32, (BR, G), 1)
            ).astype(jnp.float32)


def _phase_a1(a0, a1, table0, bseg, Wl, bl, Wr):
    def body(a0_r, a1_r, t_r, b_r, wl_r, bl_r, wr_r,
             h_r, invd_r, s1_r, s2_r, cnt_r):
        i = pl.program_id(0)
        a8 = a0_r[...] + a1_r[...]
        invd = 1.0 / jnp.maximum(a8[:, 4:5], 1.0)
        agg4 = a8[:, 0:4] * invd
        x4 = t_r[:, 0:4]
        h = (jnp.dot(agg4, wl_r[...], preferred_element_type=jnp.float32)
             + bl_r[...]
             + jnp.dot(x4, wr_r[...], preferred_element_type=jnp.float32))
        m = _onehot(b_r)
        h_r[...] = h
        invd_r[...] = invd

        @pl.when(i == 0)
        def _():
            s1_r[...] = jnp.zeros_like(s1_r)
            s2_r[...] = jnp.zeros_like(s2_r)
            cnt_r[...] = jnp.zeros_like(cnt_r)

        dn = (((0,), (0,)), ((), ()))
        s1_r[...] += lax.dot_general(m, h, dn, preferred_element_type=jnp.float32)
        s2_r[...] += lax.dot_general(m, h * h, dn, preferred_element_type=jnp.float32)
        cnt_r[...] += lax.dot_general(m, jnp.ones((BR, 1), jnp.float32), dn,
                                      preferred_element_type=jnp.float32)

    return pl.pallas_call(
        body,
        grid=(NB,),
        in_specs=[_rows((BR, 8)), _rows((BR, 8)), _rows((BR, 8)), _rows((BR, 1)),
                  _full((F_IN, H)), _full((1, H)), _full((F_IN, H))],
        out_specs=[_rows((BR, H)), _rows((BR, 1)), _full((G, H)), _full((G, H)),
                   _full((G, 1))],
        out_shape=[jax.ShapeDtypeStruct((NPAD, H), jnp.float32),
                   jax.ShapeDtypeStruct((NPAD, 1), jnp.float32),
                   jax.ShapeDtypeStruct((G, H), jnp.float32),
                   jax.ShapeDtypeStruct((G, H), jnp.float32),
                   jax.ShapeDtypeStruct((G, 1), jnp.float32)],
    )(a0, a1, table0, bseg, Wl, bl, Wr)


def _phase_a23(xq, aq, invd, bseg, Wl, bl, Wr):
    def body(x0, x1, a0, a1, iv_r, b_r, wl_r, bl_r, wr_r,
             h_r, s1_r, s2_r):
        i = pl.program_id(0)
        x = jnp.concatenate([x0[...], x1[...]], axis=1)
        agg = jnp.concatenate([a0[...], a1[...]], axis=1) * iv_r[...]
        h = (jnp.dot(agg, wl_r[...], preferred_element_type=jnp.float32)
             + bl_r[...]
             + jnp.dot(x, wr_r[...], preferred_element_type=jnp.float32))
        m = _onehot(b_r)
        h_r[...] = h

        @pl.when(i == 0)
        def _():
            s1_r[...] = jnp.zeros_like(s1_r)
            s2_r[...] = jnp.zeros_like(s2_r)

        dn = (((0,), (0,)), ((), ()))
        s1_r[...] += lax.dot_general(m, h, dn, preferred_element_type=jnp.float32)
        s2_r[...] += lax.dot_general(m, h * h, dn, preferred_element_type=jnp.float32)

    return pl.pallas_call(
        body,
        grid=(NB,),
        in_specs=[_rows((BR, 32))] * 4 + [
                  _rows((BR, 1)), _rows((BR, 1)),
                  _full((H, H)), _full((1, H)), _full((H, H))],
        out_specs=[_rows((BR, H)), _full((G, H)), _full((G, H))],
        out_shape=[jax.ShapeDtypeStruct((NPAD, H), jnp.float32),
                   jax.ShapeDtypeStruct((G, H), jnp.float32),
                   jax.ShapeDtypeStruct((G, H), jnp.float32)],
    )(*xq, *aq, invd, bseg, Wl, bl, Wr)


def _graphnorm_pq(s1, s2, cnt, gnw, gnb, gna):
    rc = 1.0 / jnp.maximum(cnt, 1.0)
    mean = s1 * rc
    ms = gna
    var = s2 * rc - (2.0 - ms) * ms * mean * mean
    inv = lax.rsqrt(var + EPS)
    p = gnw * inv
    q = gnb - p * ms * mean
    return p, q


def _phase_b(h, bseg, s1, s2, cnt, gnw, gnb, gna, resq=None):
    with_res = resq is not None

    def body(*refs):
        if with_res:
            (h_r, b_seg, s1_r, s2_r, cnt_r, w_r, b_r, a_r, r0, r1,
             o0, o1) = refs
        else:
            (h_r, b_seg, s1_r, s2_r, cnt_r, w_r, b_r, a_r,
             o0, o1) = refs
        p, q = _graphnorm_pq(s1_r[...], s2_r[...], cnt_r[...],
                             w_r[...], b_r[...], a_r[...])
        m = _onehot(b_seg)
        pb = jnp.dot(m, p, preferred_element_type=jnp.float32)
        qb = jnp.dot(m, q, preferred_element_type=jnp.float32)
        xn = pb * h_r[...] + qb
        if with_res:
            xn = xn + jnp.concatenate([r0[...], r1[...]], axis=1)
        xn = jnp.maximum(xn, 0.0)
        o0[...] = xn[:, 0:32]
        o1[...] = xn[:, 32:64]

    in_specs = [_rows((BR, H)), _rows((BR, 1)), _full((G, H)), _full((G, H)),
                _full((G, 1)), _full((1, H)), _full((1, H)), _full((1, H))]
    args = [h, bseg, s1, s2, cnt, gnw, gnb, gna]
    if with_res:
        in_specs += [_rows((BR, 32))] * 2
        args += list(resq)
    return pl.pallas_call(
        body,
        grid=(NB,),
        in_specs=in_specs,
        out_specs=[_rows((BR, 32))] * 2,
        out_shape=[jax.ShapeDtypeStruct((NPAD, 32), jnp.float32)] * 2,
    )(*args)


def _phase_b3(h, bseg, s1, s2, cnt, gnw, gnb, gna, resq, Wo, bo):
    def body(h_r, b_seg, s1_r, s2_r, cnt_r, w_r, b_r, a_r, r0, r1,
             wo_r, bo_r, out_r, pool_s):
        i = pl.program_id(0)
        p, q = _graphnorm_pq(s1_r[...], s2_r[...], cnt_r[...],
                             w_r[...], b_r[...], a_r[...])
        m = _onehot(b_seg)
        pb = jnp.dot(m, p, preferred_element_type=jnp.float32)
        qb = jnp.dot(m, q, preferred_element_type=jnp.float32)
        xn = (pb * h_r[...] + qb
              + jnp.concatenate([r0[...], r1[...]], axis=1))
        xn = jnp.maximum(xn, 0.0)

        @pl.when(i == 0)
        def _():
            pool_s[...] = jnp.zeros_like(pool_s)

        dn = (((0,), (0,)), ((), ()))
        pool_s[...] += lax.dot_general(m, xn, dn, preferred_element_type=jnp.float32)

        @pl.when(i == NB - 1)
        def _():
            pooled = pool_s[...] / cnt_r[...]
            out_r[...] = (jnp.dot(pooled, wo_r[...],
                                  preferred_element_type=jnp.float32) + bo_r[...])

    return pl.pallas_call(
        body,
        grid=(NB,),
        in_specs=[_rows((BR, H)), _rows((BR, 1)), _full((G, H)), _full((G, H)),
                  _full((G, 1)), _full((1, H)), _full((1, H)), _full((1, H))]
                 + [_rows((BR, 32))] * 2
                 + [_full((H, C)), _full((1, C))],
        out_specs=[_full((G, C))],
        out_shape=[jax.ShapeDtypeStruct((G, C), jnp.float32)],
        scratch_shapes=[pltpu.VMEM((G, H), jnp.float32)],
    )(h, bseg, s1, s2, cnt, gnw, gnb, gna, *resq, Wo, bo)[0]


def kernel(x, edge_index, batch,
           W1l, b1l, W1r, gn1_w, gn1_b, gn1_a,
           W2l, b2l, W2r, gn2_w, gn2_b, gn2_a,
           W3l, b3l, W3r, gn3_w, gn3_b, gn3_a,
           W_out, b_out):
    pad_e = E_PAD - E
    ei = jnp.stack([jnp.pad(edge_index[0], (0, pad_e)),
                    jnp.pad(edge_index[1], (0, pad_e), constant_values=N)])
    bseg = jnp.pad(batch, (0, NPAD - N), constant_values=G).reshape(NPAD, 1)
    table0 = jnp.pad(
        jnp.concatenate([x, jnp.ones((N, 1), jnp.float32)], axis=1),
        ((0, NPAD - N), (0, 8 - F_IN - 1)))
    z8 = jnp.zeros((RPT, 8), jnp.float32)
    z32 = jnp.zeros((RPT, 32), jnp.float32)

    r1 = lambda v: v.reshape(1, -1)

    a0, a1 = _sc_agg8(table0, ei, z8)
    h1, invd, s1, s2, cnt = _phase_a1(a0, a1, table0, bseg, W1l, r1(b1l), W1r)
    x1q = _phase_b(h1, bseg, s1, s2, cnt, r1(gn1_w), r1(gn1_b), r1(gn1_a))

    g1q = _sc_agg32(*x1q, ei, z32)
    h2, s1b, s2b = _phase_a23(x1q, g1q, invd, bseg, W2l, r1(b2l), W2r)
    x2q = _phase_b(h2, bseg, s1b, s2b, cnt, r1(gn2_w), r1(gn2_b), r1(gn2_a),
                   x1q)

    g2q = _sc_agg32(*x2q, ei, z32)
    h3, s1c, s2c = _phase_a23(x2q, g2q, invd, bseg, W3l, r1(b3l), W3r)
    return _phase_b3(h3, bseg, s1c, s2c, cnt, r1(gn3_w), r1(gn3_b), r1(gn3_a),
                     x2q, W_out, r1(b_out))
